# Initial kernel scaffold; baseline (speedup 1.0000x reference)
import jax
import jax.numpy as jnp
from jax.experimental import pallas as pl


def kernel(x, table):
    raise NotImplementedError("write your pallas kernel here")


# speedup vs baseline: 1.3159x; 1.3159x over previous
import functools
import math

import jax
import jax.numpy as jnp
from jax import lax
from jax.experimental import pallas as pl
from jax.experimental.pallas import tpu as pltpu
from jax.experimental.pallas import tpu_sc as plsc

D_MODEL = 768
SCALE = math.sqrt(float(D_MODEL))
LANES = 16
NC, NS = 2, 16
NW = NC * NS
B_TOTAL = 4 * 2048
BPW = B_TOTAL // NW
CHUNK = 64
NCHUNK = BPW // CHUNK
SLICES = D_MODEL // LANES


def _scale_rows(buf):
    def row_body(r, _):
        for j in range(SLICES):
            sl = pl.ds(j * LANES, LANES)
            buf[r, sl] = buf[r, sl] * SCALE
        return 0

    lax.fori_loop(0, CHUNK, row_body, 0, unroll=False)


def _emb_body(x_hbm, table_hbm, out_hbm, idx_v, buf0, buf1, sem0, sem1):
    wid = lax.axis_index("s") * NC + lax.axis_index("c")
    base = wid * BPW

    bufs = (buf0, buf1)
    sems = (sem0, sem1)

    for k in range(NCHUNK):
        pltpu.sync_copy(x_hbm.at[pl.ds(base + k * CHUNK, CHUNK)], idx_v.at[k])

    copies = [None, None]
    for k in range(NCHUNK):
        slot = k % 2
        copies[slot] = pltpu.async_copy(
            table_hbm.at[idx_v.at[k]], bufs[slot], sems[slot]
        )
        if k >= 1:
            prev = (k - 1) % 2
            copies[prev].wait()
            _scale_rows(bufs[prev])
            pltpu.sync_copy(
                bufs[prev], out_hbm.at[pl.ds(base + (k - 1) * CHUNK, CHUNK)]
            )
    last = (NCHUNK - 1) % 2
    copies[last].wait()
    _scale_rows(bufs[last])
    pltpu.sync_copy(
        bufs[last], out_hbm.at[pl.ds(base + (NCHUNK - 1) * CHUNK, CHUNK)]
    )


_emb = functools.partial(
    pl.kernel,
    out_type=jax.ShapeDtypeStruct((B_TOTAL, D_MODEL), jnp.float32),
    mesh=plsc.VectorSubcoreMesh(
        core_axis_name="c", subcore_axis_name="s", num_cores=NC, num_subcores=NS
    ),
    scratch_types=[
        pltpu.VMEM((NCHUNK, CHUNK), jnp.int32),
        pltpu.VMEM((CHUNK, D_MODEL), jnp.float32),
        pltpu.VMEM((CHUNK, D_MODEL), jnp.float32),
        pltpu.SemaphoreType.DMA,
        pltpu.SemaphoreType.DMA,
    ],
)(_emb_body)


@jax.jit
def kernel(x, table):
    x_flat = x.reshape(-1).astype(jnp.int32)
    out = _emb(x_flat, table)
    return out.reshape(x.shape + (D_MODEL,))


# speedup vs baseline: 1.3458x; 1.0228x over previous
import functools
import math

import jax
import jax.numpy as jnp
from jax import lax
from jax.experimental import pallas as pl
from jax.experimental.pallas import tpu as pltpu
from jax.experimental.pallas import tpu_sc as plsc

D_MODEL = 768
SCALE = math.sqrt(float(D_MODEL))
LANES = 16
NC, NS = 2, 16
NW = NC * NS
B_TOTAL = 4 * 2048
BPW = B_TOTAL // NW
CHUNK = 32
NCHUNK = BPW // CHUNK
NBUF = 4
PREF = 2
SLICES = D_MODEL // LANES


def _scale_rows(buf):
    def row_body(r, _):
        for j in range(SLICES):
            sl = pl.ds(j * LANES, LANES)
            buf[r, sl] = buf[r, sl] * SCALE
        return 0

    lax.fori_loop(0, CHUNK, row_body, 0, unroll=False)


def _emb_body(x_hbm, table_hbm, out_hbm, idx_v, bufs, gsems, wsems):
    wid = lax.axis_index("s") * NC + lax.axis_index("c")
    base = wid * BPW

    pltpu.sync_copy(x_hbm.at[pl.ds(base, BPW)], idx_v)

    gcopies = [None] * NBUF
    wcopies = [None] * NBUF
    for k in range(NCHUNK + PREF):
        if k < NCHUNK:
            slot = k % NBUF
            if k >= NBUF:
                wcopies[slot].wait()
            gcopies[slot] = pltpu.async_copy(
                table_hbm.at[idx_v.at[pl.ds(k * CHUNK, CHUNK)]],
                bufs[slot],
                gsems[slot],
            )
        if k >= PREF:
            j = k - PREF
            slot = j % NBUF
            gcopies[slot].wait()
            _scale_rows(bufs[slot])
            wcopies[slot] = pltpu.async_copy(
                bufs[slot],
                out_hbm.at[pl.ds(base + j * CHUNK, CHUNK)],
                wsems[slot],
            )
    for j in range(NCHUNK - NBUF, NCHUNK):
        wcopies[j % NBUF].wait()


def _emb_entry(x_hbm, table_hbm, out_hbm, *scratch):
    idx_v = scratch[0]
    bufs = scratch[1 : 1 + NBUF]
    gsems = scratch[1 + NBUF : 1 + 2 * NBUF]
    wsems = scratch[1 + 2 * NBUF : 1 + 3 * NBUF]
    _emb_body(x_hbm, table_hbm, out_hbm, idx_v, bufs, gsems, wsems)


_emb = functools.partial(
    pl.kernel,
    out_type=jax.ShapeDtypeStruct((B_TOTAL, D_MODEL), jnp.float32),
    mesh=plsc.VectorSubcoreMesh(
        core_axis_name="c", subcore_axis_name="s", num_cores=NC, num_subcores=NS
    ),
    scratch_types=(
        [pltpu.VMEM((BPW,), jnp.int32)]
        + [pltpu.VMEM((CHUNK, D_MODEL), jnp.float32) for _ in range(NBUF)]
        + [pltpu.SemaphoreType.DMA for _ in range(2 * NBUF)]
    ),
)(_emb_entry)


@jax.jit
def kernel(x, table):
    x_flat = x.reshape(-1).astype(jnp.int32)
    out = _emb(x_flat, table)
    return out.reshape(x.shape + (D_MODEL,))


# speedup vs baseline: 1.3504x; 1.0034x over previous
import functools
import math

import jax
import jax.numpy as jnp
from jax import lax
from jax.experimental import pallas as pl
from jax.experimental.pallas import tpu as pltpu
from jax.experimental.pallas import tpu_sc as plsc

D_MODEL = 768
SCALE = math.sqrt(float(D_MODEL))
LANES = 16
NC, NS = 2, 16
NW = NC * NS
B_TOTAL = 4 * 2048
BPW = B_TOTAL // NW
CHUNK = 32
NCHUNK = BPW // CHUNK
NBUF = 4
PREF = 2
SLICES = D_MODEL // LANES


def _scale_rows(buf):
    def row_body(r, _):
        for j in range(SLICES):
            sl = pl.ds(j * LANES, LANES)
            buf[r, sl] = buf[r, sl] * SCALE
        return 0

    lax.fori_loop(0, CHUNK, row_body, 0, unroll=False)


BATCH, SEQ = 4, 2048
WPB = NW // BATCH
SPW = SEQ // WPB


def _emb_body(x_hbm, table_hbm, out_hbm, idx_v, bufs, gsems, wsems):
    wid = lax.axis_index("s") * NC + lax.axis_index("c")
    brow = wid // WPB
    s0 = (wid % WPB) * SPW

    pltpu.sync_copy(x_hbm.at[brow, pl.ds(s0, SPW)], idx_v)

    gcopies = [None] * NBUF
    wcopies = [None] * NBUF
    for k in range(NCHUNK + PREF):
        if k < NCHUNK:
            slot = k % NBUF
            if k >= NBUF:
                wcopies[slot].wait()
            gcopies[slot] = pltpu.async_copy(
                table_hbm.at[idx_v.at[pl.ds(k * CHUNK, CHUNK)]],
                bufs[slot],
                gsems[slot],
            )
        if k >= PREF:
            j = k - PREF
            slot = j % NBUF
            gcopies[slot].wait()
            _scale_rows(bufs[slot])
            wcopies[slot] = pltpu.async_copy(
                bufs[slot],
                out_hbm.at[brow, pl.ds(s0 + j * CHUNK, CHUNK)],
                wsems[slot],
            )
    for j in range(NCHUNK - NBUF, NCHUNK):
        wcopies[j % NBUF].wait()


def _emb_entry(x_hbm, table_hbm, out_hbm, *scratch):
    idx_v = scratch[0]
    bufs = scratch[1 : 1 + NBUF]
    gsems = scratch[1 + NBUF : 1 + 2 * NBUF]
    wsems = scratch[1 + 2 * NBUF : 1 + 3 * NBUF]
    _emb_body(x_hbm, table_hbm, out_hbm, idx_v, bufs, gsems, wsems)


_emb = functools.partial(
    pl.kernel,
    out_type=jax.ShapeDtypeStruct((BATCH, SEQ, D_MODEL), jnp.float32),
    mesh=plsc.VectorSubcoreMesh(
        core_axis_name="c", subcore_axis_name="s", num_cores=NC, num_subcores=NS
    ),
    scratch_types=(
        [pltpu.VMEM((BPW,), jnp.int32)]
        + [pltpu.VMEM((CHUNK, D_MODEL), jnp.float32) for _ in range(NBUF)]
        + [pltpu.SemaphoreType.DMA for _ in range(2 * NBUF)]
    ),
)(_emb_entry)


@jax.jit
def kernel(x, table):
    return _emb(x.astype(jnp.int32), table)
